# baseline (speedup 1.0000x reference)
import dataclasses

import jax
import jax.numpy as jnp
from jax import lax
from jax.experimental import pallas as pl
from jax.experimental.pallas import tpu as pltpu
from jax.experimental.pallas import tpu_sc as plsc

NUM_CORES = 2
NUM_SUBCORES = 16
NW = NUM_CORES * NUM_SUBCORES
L = 16
D = 64
B = 16384
BPW = B // NW


def _sc_body(uid_hbm, iid_hbm, uemb_hbm, iemb_hbm, ubias_hbm, ibias_hbm,
             out_hbm, uid_v, iid_v, ubr_idx_v, ibr_idx_v, u_v, i_v,
             ub_v, ib_v, o_v, acc_v, sem0, sem1, sem2, sem3):
    wid = lax.axis_index("s") * NUM_CORES + lax.axis_index("c")
    base = wid * BPW

    pltpu.sync_copy(uid_hbm.at[pl.ds(base, BPW)], uid_v)
    pltpu.sync_copy(iid_hbm.at[pl.ds(base, BPW)], iid_v)

    cu = pltpu.async_copy(uemb_hbm.at[uid_v], u_v, sem0)
    ci = pltpu.async_copy(iemb_hbm.at[iid_v], i_v, sem1)

    @pl.loop(0, BPW, step=L)
    def _(o):
        ubr_idx_v[pl.ds(o, L)] = lax.shift_right_logical(
            uid_v[pl.ds(o, L)], 4)
        ibr_idx_v[pl.ds(o, L)] = lax.shift_right_logical(
            iid_v[pl.ds(o, L)], 4)

    cub = pltpu.async_copy(ubias_hbm.at[ubr_idx_v], ub_v, sem2)
    cib = pltpu.async_copy(ibias_hbm.at[ibr_idx_v], ib_v, sem3)
    cu.wait()
    ci.wait()
    cub.wait()
    cib.wait()

    iota = lax.iota(jnp.int32, L)
    fifteen = jnp.full((L,), 15, jnp.int32)

    @pl.loop(0, BPW, step=L)
    def _(g):
        for j in range(L):
            r = g + j
            acc = u_v[r, pl.ds(0, L)] * i_v[r, pl.ds(0, L)]
            for k in range(L, D, L):
                acc = acc + u_v[r, pl.ds(k, L)] * i_v[r, pl.ds(k, L)]
            acc_v[j, pl.ds(0, L)] = acc
        ublane = lax.bitwise_and(uid_v[pl.ds(g, L)], fifteen)
        iblane = lax.bitwise_and(iid_v[pl.ds(g, L)], fifteen)
        tot = plsc.load_gather(ub_v, [g + iota, ublane])
        tot = tot + plsc.load_gather(ib_v, [g + iota, iblane])
        for k in range(L):
            tot = tot + plsc.load_gather(
                acc_v, [iota, jnp.full((L,), k, jnp.int32)])
        o_v[pl.ds(g, L)] = tot

    pltpu.sync_copy(o_v, out_hbm.at[pl.ds(base, BPW)])


def kernel(user_ids, item_ids, user_emb, item_emb, user_bias, item_bias):
    uid = user_ids.astype(jnp.int32)
    iid = item_ids.astype(jnp.int32)
    nu = user_bias.shape[0]
    ni = item_bias.shape[0]
    ubias_rows = user_bias.reshape(nu // L, L)
    ibias_rows = item_bias.reshape(ni // L, L)

    mesh = plsc.VectorSubcoreMesh(core_axis_name="c", subcore_axis_name="s",
                                  num_cores=NUM_CORES,
                                  num_subcores=NUM_SUBCORES)
    cp = pltpu.CompilerParams()
    if "needs_layout_passes" in pltpu.CompilerParams.__dataclass_fields__:
        cp = dataclasses.replace(cp, needs_layout_passes=False)
    if "use_tc_tiling_on_sc" in pltpu.CompilerParams.__dataclass_fields__:
        cp = dataclasses.replace(cp, use_tc_tiling_on_sc=False)
    sc_call = pl.kernel(
        _sc_body,
        out_type=jax.ShapeDtypeStruct((B,), jnp.float32),
        mesh=mesh,
        scratch_types=[
            pltpu.VMEM((BPW,), jnp.int32),
            pltpu.VMEM((BPW,), jnp.int32),
            pltpu.VMEM((BPW,), jnp.int32),
            pltpu.VMEM((BPW,), jnp.int32),
            pltpu.VMEM((BPW, D), jnp.float32),
            pltpu.VMEM((BPW, D), jnp.float32),
            pltpu.VMEM((BPW, L), jnp.float32),
            pltpu.VMEM((BPW, L), jnp.float32),
            pltpu.VMEM((BPW,), jnp.float32),
            pltpu.VMEM((L, L), jnp.float32),
            pltpu.SemaphoreType.DMA,
            pltpu.SemaphoreType.DMA,
            pltpu.SemaphoreType.DMA,
            pltpu.SemaphoreType.DMA,
        ],
        compiler_params=cp,
    )
    return sc_call(uid, iid, user_emb, item_emb, ubias_rows, ibias_rows)
